# Initial kernel scaffold; baseline (speedup 1.0000x reference)
import functools

import jax
import jax.numpy as jnp
from jax import lax
from jax.experimental import pallas as pl
from jax.experimental.pallas import tpu as pltpu
from jax.experimental.pallas import tpu_sc as plsc

B = 16384
SKILL_LEN = 50
NC, NS = 2, 16
NW = NC * NS
BPW = B // NW
SK_PER_W = BPW * SKILL_LEN
SK_CHUNK = 1024

_vector_mesh = plsc.VectorSubcoreMesh(core_axis_name="c", subcore_axis_name="s")


def _sc_gather_body(job_hbm, cat_hbm, loc_hbm, lev_hbm, tok_hbm,
                    job_t, cat_t, loc_t, lev_t, skill_t,
                    ejob_hbm, ecat_hbm, eloc_hbm, elev_hbm, esk_hbm,
                    idx_v, rows32_v, rows64_v, tokidx_v, srows_v, sem):
    wid = lax.axis_index("s") * NC + lax.axis_index("c")
    base = wid * BPW

    pltpu.sync_copy(job_hbm.at[pl.ds(base, BPW)], idx_v)
    pltpu.async_copy(job_t.at[idx_v], rows32_v, sem).wait()
    pltpu.sync_copy(rows32_v, ejob_hbm.at[pl.ds(base, BPW)])
    pltpu.sync_copy(loc_hbm.at[pl.ds(base, BPW)], idx_v)
    pltpu.async_copy(loc_t.at[idx_v], rows32_v, sem).wait()
    pltpu.sync_copy(rows32_v, eloc_hbm.at[pl.ds(base, BPW)])
    pltpu.sync_copy(lev_hbm.at[pl.ds(base, BPW)], idx_v)
    pltpu.async_copy(lev_t.at[idx_v], rows32_v, sem).wait()
    pltpu.sync_copy(rows32_v, elev_hbm.at[pl.ds(base, BPW)])
    pltpu.sync_copy(cat_hbm.at[pl.ds(base, BPW)], idx_v)
    pltpu.async_copy(cat_t.at[idx_v], rows64_v, sem).wait()
    pltpu.sync_copy(rows64_v, ecat_hbm.at[pl.ds(base, BPW)])

    sbase = wid * SK_PER_W

    @pl.loop(0, SK_PER_W, step=SK_CHUNK)
    def _(i):
        pltpu.sync_copy(tok_hbm.at[pl.ds(sbase + i, SK_CHUNK)], tokidx_v)
        pltpu.async_copy(skill_t.at[tokidx_v], srows_v, sem).wait()
        pltpu.sync_copy(srows_v, esk_hbm.at[pl.ds(sbase + i, SK_CHUNK)])


@jax.jit
def _sc_gather(job_id, category, location, level, tok_flat,
               job_t, cat_t, loc_t, lev_t, skill_t):
    f32 = jnp.float32
    out_type = (
        jax.ShapeDtypeStruct((B, 32), f32),
        jax.ShapeDtypeStruct((B, 64), f32),
        jax.ShapeDtypeStruct((B, 32), f32),
        jax.ShapeDtypeStruct((B, 32), f32),
        jax.ShapeDtypeStruct((B * SKILL_LEN, 32), f32),
    )
    scratch = [
        pltpu.VMEM((BPW,), jnp.int32),
        pltpu.VMEM((BPW, 32), f32),
        pltpu.VMEM((BPW, 64), f32),
        pltpu.VMEM((SK_CHUNK,), jnp.int32),
        pltpu.VMEM((SK_CHUNK, 32), f32),
        pltpu.SemaphoreType.DMA,
    ]
    k = pl.kernel(_sc_gather_body, out_type=out_type, mesh=_vector_mesh,
                  scratch_types=scratch)
    return k(job_id, category, location, level, tok_flat,
             job_t, cat_t, loc_t, lev_t, skill_t)


BB = 512


def _mlp_body(ejob, ecat, eloc, elev, esk, W1, b1, W2, b2, W3, b3, out):
    sk = esk[...]
    acc = sk[:, 0:32]
    for j in range(1, SKILL_LEN):
        acc = acc + sk[:, j * 32:(j + 1) * 32]
    skm = acc * (1.0 / SKILL_LEN)
    feat = jnp.concatenate(
        [ejob[...], ecat[...], eloc[...], elev[...], skm], axis=1)
    h = jnp.maximum(jnp.dot(feat, W1[...],
                            preferred_element_type=jnp.float32) + b1[...], 0.0)
    h = jnp.maximum(jnp.dot(h, W2[...],
                            preferred_element_type=jnp.float32) + b2[...], 0.0)
    out[...] = jnp.dot(h, W3[...],
                       preferred_element_type=jnp.float32) + b3[...]


@jax.jit
def _tc_mlp(ejob, ecat, eloc, elev, esk2d, W1, b1, W2, b2, W3, b3):
    nb = B // BB
    bspec = lambda c: pl.BlockSpec((BB, c), lambda i: (i, 0))
    full = lambda a: pl.BlockSpec(a.shape, lambda i: tuple(0 for _ in a.shape))
    return pl.pallas_call(
        _mlp_body,
        grid=(nb,),
        in_specs=[bspec(32), bspec(64), bspec(32), bspec(32),
                  bspec(SKILL_LEN * 32),
                  full(W1), full(b1), full(W2), full(b2), full(W3), full(b3)],
        out_specs=pl.BlockSpec((BB, 64), lambda i: (i, 0)),
        out_shape=jax.ShapeDtypeStruct((B, 64), jnp.float32),
    )(ejob, ecat, eloc, elev, esk2d, W1, b1, W2, b2, W3, b3)


def kernel(job_id, category, location, level, skill_tokens,
           job_table, category_table, location_table, level_table, skill_table,
           W1, b1, W2, b2, W3, b3):
    tok_flat = skill_tokens.reshape(-1)
    ejob, ecat, eloc, elev, esk = _sc_gather(
        job_id, category, location, level, tok_flat,
        job_table, category_table, location_table, level_table, skill_table)
    esk2d = esk.reshape(B, SKILL_LEN * 32)
    return _tc_mlp(ejob, ecat, eloc, elev, esk2d, W1, b1, W2, b2, W3, b3)


# speedup vs baseline: 7.9101x; 7.9101x over previous
import functools

import jax
import jax.numpy as jnp
from jax import lax
from jax.experimental import pallas as pl
from jax.experimental.pallas import tpu as pltpu
from jax.experimental.pallas import tpu_sc as plsc

B = 16384
SKILL_LEN = 50
NC, NS = 2, 16
NW = NC * NS
BPW = B // NW
SK_PER_W = BPW * SKILL_LEN
SK_CHUNK = 1024

def _sc_gather_body(job_hbm, cat_hbm, loc_hbm, lev_hbm, tok_hbm,
                    job_t, cat_t, loc_t, lev_t, skill_t,
                    ejob_hbm, ecat_hbm, eloc_hbm, elev_hbm, esk_hbm,
                    idx_v, rows32_v, rows64_v, tokidx_v, srows_v, sem):
    wid = lax.axis_index("s") * NC + lax.axis_index("c")
    base = wid * BPW

    pltpu.sync_copy(job_hbm.at[pl.ds(base, BPW)], idx_v)
    pltpu.async_copy(job_t.at[idx_v], rows32_v, sem).wait()
    pltpu.sync_copy(rows32_v, ejob_hbm.at[pl.ds(base, BPW)])
    pltpu.sync_copy(loc_hbm.at[pl.ds(base, BPW)], idx_v)
    pltpu.async_copy(loc_t.at[idx_v], rows32_v, sem).wait()
    pltpu.sync_copy(rows32_v, eloc_hbm.at[pl.ds(base, BPW)])
    pltpu.sync_copy(lev_hbm.at[pl.ds(base, BPW)], idx_v)
    pltpu.async_copy(lev_t.at[idx_v], rows32_v, sem).wait()
    pltpu.sync_copy(rows32_v, elev_hbm.at[pl.ds(base, BPW)])
    pltpu.sync_copy(cat_hbm.at[pl.ds(base, BPW)], idx_v)
    pltpu.async_copy(cat_t.at[idx_v], rows64_v, sem).wait()
    pltpu.sync_copy(rows64_v, ecat_hbm.at[pl.ds(base, BPW)])

    sbase = wid * SK_PER_W

    @pl.loop(0, SK_PER_W, step=SK_CHUNK)
    def _(i):
        pltpu.sync_copy(tok_hbm.at[pl.ds(sbase + i, SK_CHUNK)], tokidx_v)
        pltpu.async_copy(skill_t.at[tokidx_v], srows_v, sem).wait()
        pltpu.sync_copy(srows_v, esk_hbm.at[pl.ds(sbase + i, SK_CHUNK)])


@jax.jit
def _sc_gather(job_id, category, location, level, tok_flat,
               job_t, cat_t, loc_t, lev_t, skill_t):
    f32 = jnp.float32
    out_type = (
        jax.ShapeDtypeStruct((B, 32), f32),
        jax.ShapeDtypeStruct((B, 64), f32),
        jax.ShapeDtypeStruct((B, 32), f32),
        jax.ShapeDtypeStruct((B, 32), f32),
        jax.ShapeDtypeStruct((B * SKILL_LEN, 32), f32),
    )
    scratch = [
        pltpu.VMEM((BPW,), jnp.int32),
        pltpu.VMEM((BPW, 32), f32),
        pltpu.VMEM((BPW, 64), f32),
        pltpu.VMEM((SK_CHUNK,), jnp.int32),
        pltpu.VMEM((SK_CHUNK, 32), f32),
        pltpu.SemaphoreType.DMA,
    ]
    mesh = plsc.VectorSubcoreMesh(core_axis_name="c", subcore_axis_name="s")
    k = pl.kernel(_sc_gather_body, out_type=out_type, mesh=mesh,
                  scratch_types=scratch,
                  compiler_params=pltpu.CompilerParams(
                      use_tc_tiling_on_sc=False))
    return k(job_id, category, location, level, tok_flat,
             job_t, cat_t, loc_t, lev_t, skill_t)


BB = 512


def _mlp_body(ejob, ecat, eloc, elev, esk, W1, b1, W2, b2, W3, b3, out):
    sk = esk[...]
    acc = sk[:, 0:32]
    for j in range(1, SKILL_LEN):
        acc = acc + sk[:, j * 32:(j + 1) * 32]
    skm = acc * (1.0 / SKILL_LEN)
    feat = jnp.concatenate(
        [ejob[...], ecat[...], eloc[...], elev[...], skm], axis=1)
    h = jnp.maximum(jnp.dot(feat, W1[...],
                            preferred_element_type=jnp.float32) + b1[...], 0.0)
    h = jnp.maximum(jnp.dot(h, W2[...],
                            preferred_element_type=jnp.float32) + b2[...], 0.0)
    out[...] = jnp.dot(h, W3[...],
                       preferred_element_type=jnp.float32) + b3[...]


@jax.jit
def _tc_mlp(ejob, ecat, eloc, elev, esk2d, W1, b1, W2, b2, W3, b3):
    nb = B // BB
    bspec = lambda c: pl.BlockSpec((BB, c), lambda i: (i, 0))
    full = lambda a: pl.BlockSpec(a.shape, lambda i: tuple(0 for _ in a.shape))
    return pl.pallas_call(
        _mlp_body,
        grid=(nb,),
        in_specs=[bspec(32), bspec(64), bspec(32), bspec(32),
                  bspec(SKILL_LEN * 32),
                  full(W1), full(b1), full(W2), full(b2), full(W3), full(b3)],
        out_specs=pl.BlockSpec((BB, 64), lambda i: (i, 0)),
        out_shape=jax.ShapeDtypeStruct((B, 64), jnp.float32),
    )(ejob, ecat, eloc, elev, esk2d, W1, b1, W2, b2, W3, b3)


def kernel(job_id, category, location, level, skill_tokens,
           job_table, category_table, location_table, level_table, skill_table,
           W1, b1, W2, b2, W3, b3):
    tok_flat = skill_tokens.reshape(-1)
    ejob, ecat, eloc, elev, esk = _sc_gather(
        job_id, category, location, level, tok_flat,
        job_table, category_table, location_table, level_table, skill_table)
    esk2d = esk.reshape(B, SKILL_LEN * 32)
    return _tc_mlp(ejob, ecat, eloc, elev, esk2d, W1, b1, W2, b2, W3, b3)


# speedup vs baseline: 12.1875x; 1.5407x over previous
import functools

import jax
import jax.numpy as jnp
from jax import lax
from jax.experimental import pallas as pl
from jax.experimental.pallas import tpu as pltpu
from jax.experimental.pallas import tpu_sc as plsc

B = 16384
SKILL_LEN = 50
FEAT = 192
NC, NS = 2, 16
NW = NC * NS
BPW = B // NW
SK_PER_W = BPW * SKILL_LEN
SK_CHUNK = 1024


def _sc_gather_body(job_hbm, cat_hbm, loc_hbm, lev_hbm, tok_hbm, lid_hbm,
                    zero_hbm, job_t, cat_t, loc_t, lev_t, skill_t,
                    feat_hbm,
                    idx_v, rows32_v, rows64_v, tokidx_v, lid_v, srows_v,
                    acc_sh, sem):
    sid = lax.axis_index("s")
    wid = sid * NC + lax.axis_index("c")
    base = wid * BPW
    rows = pl.ds(base, BPW)
    acc_v = acc_sh.at[sid]

    pltpu.sync_copy(job_hbm.at[rows], idx_v)
    pltpu.async_copy(job_t.at[idx_v], rows32_v, sem).wait()
    pltpu.sync_copy(rows32_v, feat_hbm.at[rows, pl.ds(0, 32)])
    pltpu.sync_copy(cat_hbm.at[rows], idx_v)
    pltpu.async_copy(cat_t.at[idx_v], rows64_v, sem).wait()
    pltpu.sync_copy(rows64_v, feat_hbm.at[rows, pl.ds(32, 64)])
    pltpu.sync_copy(loc_hbm.at[rows], idx_v)
    pltpu.async_copy(loc_t.at[idx_v], rows32_v, sem).wait()
    pltpu.sync_copy(rows32_v, feat_hbm.at[rows, pl.ds(96, 32)])
    pltpu.sync_copy(lev_hbm.at[rows], idx_v)
    pltpu.async_copy(lev_t.at[idx_v], rows32_v, sem).wait()
    pltpu.sync_copy(rows32_v, feat_hbm.at[rows, pl.ds(128, 32)])

    pltpu.sync_copy(zero_hbm, acc_v)
    sbase = wid * SK_PER_W

    @pl.loop(0, SK_PER_W, step=SK_CHUNK)
    def _(i):
        pltpu.sync_copy(tok_hbm.at[pl.ds(sbase + i, SK_CHUNK)], tokidx_v)
        pltpu.sync_copy(lid_hbm.at[pl.ds(i, SK_CHUNK)], lid_v)
        pltpu.async_copy(skill_t.at[tokidx_v], srows_v, sem).wait()
        pltpu.async_copy(srows_v, acc_v.at[lid_v], sem, add=True).wait()

    pltpu.sync_copy(acc_v, feat_hbm.at[rows, pl.ds(160, 32)])


@jax.jit
def _sc_gather(job_id, category, location, level, tok_flat, local_ids, zeros,
               job_t, cat_t, loc_t, lev_t, skill_t):
    f32 = jnp.float32
    out_type = jax.ShapeDtypeStruct((B, FEAT), f32)
    scratch = [
        pltpu.VMEM((BPW,), jnp.int32),
        pltpu.VMEM((BPW, 32), f32),
        pltpu.VMEM((BPW, 64), f32),
        pltpu.VMEM((SK_CHUNK,), jnp.int32),
        pltpu.VMEM((SK_CHUNK,), jnp.int32),
        pltpu.VMEM((SK_CHUNK, 32), f32),
        pltpu.VMEM_SHARED((NS, BPW, 32), f32),
        pltpu.SemaphoreType.DMA,
    ]
    mesh = plsc.VectorSubcoreMesh(core_axis_name="c", subcore_axis_name="s")
    k = pl.kernel(_sc_gather_body, out_type=out_type, mesh=mesh,
                  scratch_types=scratch,
                  compiler_params=pltpu.CompilerParams(
                      use_tc_tiling_on_sc=False))
    return k(job_id, category, location, level, tok_flat, local_ids, zeros,
             job_t, cat_t, loc_t, lev_t, skill_t)


BB = 512


def _mlp_body(feat, W1, b1, W2, b2, W3, b3, out):
    h = jnp.maximum(jnp.dot(feat[...], W1[...],
                            preferred_element_type=jnp.float32) + b1[...], 0.0)
    h = jnp.maximum(jnp.dot(h, W2[...],
                            preferred_element_type=jnp.float32) + b2[...], 0.0)
    out[...] = jnp.dot(h, W3[...],
                       preferred_element_type=jnp.float32) + b3[...]


@jax.jit
def _tc_mlp(feat, W1, b1, W2, b2, W3, b3):
    nb = B // BB
    full = lambda a: pl.BlockSpec(a.shape, lambda i: tuple(0 for _ in a.shape))
    return pl.pallas_call(
        _mlp_body,
        grid=(nb,),
        in_specs=[pl.BlockSpec((BB, FEAT), lambda i: (i, 0)),
                  full(W1), full(b1), full(W2), full(b2), full(W3), full(b3)],
        out_specs=pl.BlockSpec((BB, 64), lambda i: (i, 0)),
        out_shape=jax.ShapeDtypeStruct((B, 64), jnp.float32),
    )(feat, W1, b1, W2, b2, W3, b3)


def kernel(job_id, category, location, level, skill_tokens,
           job_table, category_table, location_table, level_table, skill_table,
           W1, b1, W2, b2, W3, b3):
    tok_flat = skill_tokens.reshape(-1)
    local_ids = jnp.arange(SK_PER_W, dtype=jnp.int32) // SKILL_LEN
    zeros = jnp.zeros((BPW, 32), jnp.float32)
    feat = _sc_gather(job_id, category, location, level, tok_flat, local_ids,
                      zeros, job_table, category_table, location_table,
                      level_table, skill_table)
    scale = jnp.concatenate([jnp.ones((160,), jnp.float32),
                             jnp.full((32,), 1.0 / SKILL_LEN, jnp.float32)])
    W1s = W1 * scale[:, None]
    return _tc_mlp(feat, W1s, b1, W2, b2, W3, b3)


# speedup vs baseline: 12.3449x; 1.0129x over previous
import functools

import jax
import jax.numpy as jnp
from jax import lax
from jax.experimental import pallas as pl
from jax.experimental.pallas import tpu as pltpu
from jax.experimental.pallas import tpu_sc as plsc

B = 16384
SKILL_LEN = 50
FEAT = 192
NC, NS = 2, 16
NW = NC * NS
BPW = B // NW
SPC = 8
SK_CHUNK = SPC * SKILL_LEN
NCHUNK = BPW // SPC


def _sc_gather_body(job_hbm, cat_hbm, loc_hbm, lev_hbm, tok_hbm, lid_hbm,
                    zero_hbm, job_t, cat_t, loc_t, lev_t, skill_t,
                    feat_hbm,
                    idxa, idxb, r32a, r32b, r64,
                    tok0, tok1, lid0, lid1, sr0, sr1,
                    acc_sh, gsem0, gsem1, ssem0, ssem1, wsem, zsem):
    sid = lax.axis_index("s")
    wid = sid * NC + lax.axis_index("c")
    base = wid * BPW
    rows = pl.ds(base, BPW)
    acc_v = acc_sh.at[sid]

    zcp = pltpu.async_copy(zero_hbm, acc_v, zsem)

    pltpu.sync_copy(job_hbm.at[rows], idxa)
    g_job = pltpu.async_copy(job_t.at[idxa], r32a, gsem0)
    pltpu.sync_copy(loc_hbm.at[rows], idxb)
    g_loc = pltpu.async_copy(loc_t.at[idxb], r32b, gsem1)
    g_job.wait()
    w_job = pltpu.async_copy(r32a, feat_hbm.at[rows, pl.ds(0, 32)], wsem)
    g_loc.wait()
    w_loc = pltpu.async_copy(r32b, feat_hbm.at[rows, pl.ds(96, 32)], wsem)
    pltpu.sync_copy(cat_hbm.at[rows], idxa)
    g_cat = pltpu.async_copy(cat_t.at[idxa], r64, gsem0)
    pltpu.sync_copy(lev_hbm.at[rows], idxb)
    w_job.wait()
    g_lev = pltpu.async_copy(lev_t.at[idxb], r32a, gsem1)
    g_cat.wait()
    w_cat = pltpu.async_copy(r64, feat_hbm.at[rows, pl.ds(32, 64)], wsem)
    g_lev.wait()
    w_lev = pltpu.async_copy(r32a, feat_hbm.at[rows, pl.ds(128, 32)], wsem)

    sbase = wid * BPW

    def load_idx(c, tok_v, lid_v):
        pltpu.sync_copy(tok_hbm.at[pl.ds((sbase + c * SPC) * SKILL_LEN,
                                         SK_CHUNK)], tok_v)
        pltpu.sync_copy(lid_hbm.at[pl.ds(c * SK_CHUNK, SK_CHUNK)], lid_v)

    def start_gather(tok_v, sr_v, sem):
        return pltpu.async_copy(skill_t.at[tok_v], sr_v, sem)

    def start_scat(sr_v, lid_v, sem):
        return pltpu.async_copy(sr_v, acc_v.at[lid_v], sem, add=True)

    load_idx(0, tok0, lid0)
    start_gather(tok0, sr0, gsem0)
    load_idx(1, tok1, lid1)
    start_gather(tok1, sr1, gsem1)
    zcp.wait()

    def dummy_g(sem):
        return pltpu.make_async_copy(skill_t.at[tok0], sr0, sem)

    def dummy_s(sem):
        return pltpu.make_async_copy(sr0, acc_v.at[lid0], sem)

    @pl.loop(0, NCHUNK - 2, step=2)
    def _(c):
        dummy_g(gsem0).wait()
        start_scat(sr0, lid0, ssem0)
        dummy_g(gsem1).wait()
        start_scat(sr1, lid1, ssem1)
        dummy_s(ssem0).wait()
        load_idx(c + 2, tok0, lid0)
        start_gather(tok0, sr0, gsem0)
        dummy_s(ssem1).wait()
        load_idx(c + 3, tok1, lid1)
        start_gather(tok1, sr1, gsem1)

    dummy_g(gsem0).wait()
    start_scat(sr0, lid0, ssem0)
    dummy_g(gsem1).wait()
    start_scat(sr1, lid1, ssem1)
    dummy_s(ssem0).wait()
    dummy_s(ssem1).wait()

    pltpu.sync_copy(acc_v, feat_hbm.at[rows, pl.ds(160, 32)])
    w_loc.wait()
    w_cat.wait()
    w_lev.wait()


@jax.jit
def _sc_gather(job_id, category, location, level, tok2d, local_ids, zeros,
               job_t, cat_t, loc_t, lev_t, skill_t):
    f32 = jnp.float32
    i32 = jnp.int32
    out_type = jax.ShapeDtypeStruct((B, FEAT), f32)
    scratch = [
        pltpu.VMEM((BPW,), i32),
        pltpu.VMEM((BPW,), i32),
        pltpu.VMEM((BPW, 32), f32),
        pltpu.VMEM((BPW, 32), f32),
        pltpu.VMEM((BPW, 64), f32),
        pltpu.VMEM((SK_CHUNK,), i32),
        pltpu.VMEM((SK_CHUNK,), i32),
        pltpu.VMEM((SK_CHUNK,), i32),
        pltpu.VMEM((SK_CHUNK,), i32),
        pltpu.VMEM((SK_CHUNK, 32), f32),
        pltpu.VMEM((SK_CHUNK, 32), f32),
        pltpu.VMEM_SHARED((NS, BPW, 32), f32),
        pltpu.SemaphoreType.DMA,
        pltpu.SemaphoreType.DMA,
        pltpu.SemaphoreType.DMA,
        pltpu.SemaphoreType.DMA,
        pltpu.SemaphoreType.DMA,
        pltpu.SemaphoreType.DMA,
    ]
    mesh = plsc.VectorSubcoreMesh(core_axis_name="c", subcore_axis_name="s")
    k = pl.kernel(_sc_gather_body, out_type=out_type, mesh=mesh,
                  scratch_types=scratch,
                  compiler_params=pltpu.CompilerParams(
                      use_tc_tiling_on_sc=False))
    return k(job_id, category, location, level, tok2d, local_ids, zeros,
             job_t, cat_t, loc_t, lev_t, skill_t)


BB = 512


def _mlp_body(feat, W1, b1, W2, b2, W3, b3, out):
    h = jnp.maximum(jnp.dot(feat[...], W1[...],
                            preferred_element_type=jnp.float32) + b1[...], 0.0)
    h = jnp.maximum(jnp.dot(h, W2[...],
                            preferred_element_type=jnp.float32) + b2[...], 0.0)
    out[...] = jnp.dot(h, W3[...],
                       preferred_element_type=jnp.float32) + b3[...]


@jax.jit
def _tc_mlp(feat, W1, b1, W2, b2, W3, b3):
    nb = B // BB
    full = lambda a: pl.BlockSpec(a.shape, lambda i: tuple(0 for _ in a.shape))
    return pl.pallas_call(
        _mlp_body,
        grid=(nb,),
        in_specs=[pl.BlockSpec((BB, FEAT), lambda i: (i, 0)),
                  full(W1), full(b1), full(W2), full(b2), full(W3), full(b3)],
        out_specs=pl.BlockSpec((BB, 64), lambda i: (i, 0)),
        out_shape=jax.ShapeDtypeStruct((B, 64), jnp.float32),
    )(feat, W1, b1, W2, b2, W3, b3)


def kernel(job_id, category, location, level, skill_tokens,
           job_table, category_table, location_table, level_table, skill_table,
           W1, b1, W2, b2, W3, b3):
    local_ids = jnp.arange(BPW * SKILL_LEN, dtype=jnp.int32) // SKILL_LEN
    zeros = jnp.zeros((BPW, 32), jnp.float32)
    feat = _sc_gather(job_id, category, location, level,
                      skill_tokens.reshape(-1), local_ids, zeros,
                      job_table, category_table, location_table,
                      level_table, skill_table)
    scale = jnp.concatenate([jnp.ones((160,), jnp.float32),
                             jnp.full((32,), 1.0 / SKILL_LEN, jnp.float32)])
    W1s = W1 * scale[:, None]
    return _tc_mlp(feat, W1s, b1, W2, b2, W3, b3)
